# baseline (speedup 1.0000x reference)
import functools

import jax
import jax.numpy as jnp
from jax.experimental import pallas as pl
from jax.experimental.pallas import tpu as pltpu

_N = 10000
_E = 320000
_DIN = 128
_DH = 256
_DZ = 64
_NCL = 16
_ATT = 16
_ALPHA = 0.01

_ROWS = 2000


def _elu(x):
    return jnp.where(x > 0, x, jnp.exp(jnp.minimum(x, 0.0)) - 1.0)


def _normalize(x):
    n = jnp.sqrt(jnp.sum(x * x, axis=1, keepdims=True))
    return x / jnp.maximum(n, 1e-12)


def _mm1_body(data_ref, n1_ref, n2_ref, w1_ref, o1_ref, o2_ref):
    w1 = w1_ref[...]
    f1 = data_ref[...] + _ALPHA * n1_ref[...]
    f2 = data_ref[...] + _ALPHA * n2_ref[...]
    o1_ref[...] = jnp.dot(f1, w1, preferred_element_type=jnp.float32)
    o2_ref[...] = jnp.dot(f2, w1, preferred_element_type=jnp.float32)


def _mm1(data, noise1, noise2, W1):
    grid = (_N // _ROWS,)
    return pl.pallas_call(
        _mm1_body,
        grid=grid,
        in_specs=[
            pl.BlockSpec((_ROWS, _DIN), lambda i: (i, 0)),
            pl.BlockSpec((_ROWS, _DIN), lambda i: (i, 0)),
            pl.BlockSpec((_ROWS, _DIN), lambda i: (i, 0)),
            pl.BlockSpec((_DIN, _DH), lambda i: (0, 0)),
        ],
        out_specs=[
            pl.BlockSpec((_ROWS, _DH), lambda i: (i, 0)),
            pl.BlockSpec((_ROWS, _DH), lambda i: (i, 0)),
        ],
        out_shape=[
            jax.ShapeDtypeStruct((_N, _DH), jnp.float32),
            jax.ShapeDtypeStruct((_N, _DH), jnp.float32),
        ],
    )(data, noise1, noise2, W1)


def _mm2_body(s1_ref, s2_ref, w2_ref, o1_ref, o2_ref):
    w2 = w2_ref[...]
    o1_ref[...] = jnp.dot(_elu(s1_ref[...]), w2, preferred_element_type=jnp.float32)
    o2_ref[...] = jnp.dot(_elu(s2_ref[...]), w2, preferred_element_type=jnp.float32)


def _mm2(s1, s2, W2):
    grid = (_N // _ROWS,)
    return pl.pallas_call(
        _mm2_body,
        grid=grid,
        in_specs=[
            pl.BlockSpec((_ROWS, _DH), lambda i: (i, 0)),
            pl.BlockSpec((_ROWS, _DH), lambda i: (i, 0)),
            pl.BlockSpec((_DH, _DZ), lambda i: (0, 0)),
        ],
        out_specs=[
            pl.BlockSpec((_ROWS, _DZ), lambda i: (i, 0)),
            pl.BlockSpec((_ROWS, _DZ), lambda i: (i, 0)),
        ],
        out_shape=[
            jax.ShapeDtypeStruct((_N, _DZ), jnp.float32),
            jax.ShapeDtypeStruct((_N, _DZ), jnp.float32),
        ],
    )(s1, s2, W2)


def _heads_body(s1_ref, s2_ref, w1_ref, w2_ref,
                aw1_ref, ab1_ref, aw2_ref,
                iw1_ref, ib1_ref, iw2_ref, ib2_ref,
                cw1_ref, cb1_ref, cw2_ref, cb2_ref,
                h1_ref, h2_ref, z1_ref, z2_ref, z_ref,
                l1_ref, l2_ref, xrec_ref):
    z1 = _normalize(_elu(s1_ref[...]))
    z2 = _normalize(_elu(s2_ref[...]))
    z1_ref[...] = z1
    z2_ref[...] = z2

    iw1 = iw1_ref[...]
    ib1 = ib1_ref[...]
    iw2 = iw2_ref[...]
    ib2 = ib2_ref[...]

    def ins_head(z):
        t = jnp.maximum(jnp.dot(z, iw1, preferred_element_type=jnp.float32) + ib1[None, :], 0.0)
        return jnp.maximum(jnp.dot(t, iw2, preferred_element_type=jnp.float32) + ib2[None, :], 0.0)

    h1_ref[...] = _normalize(ins_head(z1))
    h2_ref[...] = _normalize(ins_head(z2))

    cw1 = cw1_ref[...]
    cb1 = cb1_ref[...]
    cw2 = cw2_ref[...]
    cb2 = cb2_ref[...]

    def cls_head(z):
        t = jnp.maximum(jnp.dot(z, cw1, preferred_element_type=jnp.float32) + cb1[None, :], 0.0)
        logit = jnp.dot(t, cw2, preferred_element_type=jnp.float32) + cb2[None, :]
        m = jnp.max(logit, axis=1, keepdims=True)
        e = jnp.exp(logit - m)
        return e / jnp.sum(e, axis=1, keepdims=True)

    l1_ref[...] = cls_head(z1)
    l2_ref[...] = cls_head(z2)

    aw1 = aw1_ref[...]
    ab1 = ab1_ref[...]
    aw2 = aw2_ref[...]
    a1 = jnp.dot(jnp.tanh(jnp.dot(z1, aw1, preferred_element_type=jnp.float32) + ab1[None, :]),
                 aw2, preferred_element_type=jnp.float32)
    a2 = jnp.dot(jnp.tanh(jnp.dot(z2, aw1, preferred_element_type=jnp.float32) + ab1[None, :]),
                 aw2, preferred_element_type=jnp.float32)
    m = jnp.maximum(a1, a2)
    e1 = jnp.exp(a1 - m)
    e2 = jnp.exp(a2 - m)
    denom = e1 + e2
    z = (e1 / denom) * z1 + (e2 / denom) * z2
    z_ref[...] = z

    w1 = w1_ref[...]
    w2 = w2_ref[...]
    t = jnp.maximum(
        jax.lax.dot_general(z, w2, (((1,), (1,)), ((), ())),
                            preferred_element_type=jnp.float32), 0.0)
    xrec_ref[...] = jax.lax.dot_general(t, w1, (((1,), (1,)), ((), ())),
                                        preferred_element_type=jnp.float32)


def _heads(s1, s2, W1, W2, aw1, ab1, aw2, iw1, ib1, iw2, ib2, cw1, cb1, cw2, cb2):
    grid = (_N // _ROWS,)
    row = lambda i: (i, 0)
    zero2 = lambda i: (0, 0)
    zero1 = lambda i: (0,)
    return pl.pallas_call(
        _heads_body,
        grid=grid,
        in_specs=[
            pl.BlockSpec((_ROWS, _DZ), row),
            pl.BlockSpec((_ROWS, _DZ), row),
            pl.BlockSpec((_DIN, _DH), zero2),
            pl.BlockSpec((_DH, _DZ), zero2),
            pl.BlockSpec((_DZ, _ATT), zero2),
            pl.BlockSpec((_ATT,), zero1),
            pl.BlockSpec((_ATT, 1), zero2),
            pl.BlockSpec((_DZ, _DZ), zero2),
            pl.BlockSpec((_DZ,), zero1),
            pl.BlockSpec((_DZ, _DZ), zero2),
            pl.BlockSpec((_DZ,), zero1),
            pl.BlockSpec((_DZ, _DZ), zero2),
            pl.BlockSpec((_DZ,), zero1),
            pl.BlockSpec((_DZ, _NCL), zero2),
            pl.BlockSpec((_NCL,), zero1),
        ],
        out_specs=[
            pl.BlockSpec((_ROWS, _DZ), row),
            pl.BlockSpec((_ROWS, _DZ), row),
            pl.BlockSpec((_ROWS, _DZ), row),
            pl.BlockSpec((_ROWS, _DZ), row),
            pl.BlockSpec((_ROWS, _DZ), row),
            pl.BlockSpec((_ROWS, _NCL), row),
            pl.BlockSpec((_ROWS, _NCL), row),
            pl.BlockSpec((_ROWS, _DIN), row),
        ],
        out_shape=[
            jax.ShapeDtypeStruct((_N, _DZ), jnp.float32),
            jax.ShapeDtypeStruct((_N, _DZ), jnp.float32),
            jax.ShapeDtypeStruct((_N, _DZ), jnp.float32),
            jax.ShapeDtypeStruct((_N, _DZ), jnp.float32),
            jax.ShapeDtypeStruct((_N, _DZ), jnp.float32),
            jax.ShapeDtypeStruct((_N, _NCL), jnp.float32),
            jax.ShapeDtypeStruct((_N, _NCL), jnp.float32),
            jax.ShapeDtypeStruct((_N, _DIN), jnp.float32),
        ],
    )(s1, s2, W1, W2, aw1, ab1, aw2, iw1, ib1, iw2, ib2, cw1, cb1, cw2, cb2)


def _spmm(idx, vals, X):
    gathered = X[idx[1]] * vals[:, None]
    return jax.ops.segment_sum(gathered, idx[0], num_segments=_N)


def kernel(data, adj1_indices, adj1_values, adj2_indices, adj2_values,
           W1, W2, att_w1, att_b1, att_w2,
           ins_w1, ins_b1, ins_w2, ins_b2,
           cls_w1, cls_b1, cls_w2, cls_b2):
    noise1 = jax.random.normal(jax.random.key(101), data.shape, jnp.float32)
    noise2 = jax.random.normal(jax.random.key(202), data.shape, jnp.float32)

    t1_1, t1_2 = _mm1(data, noise1, noise2, W1)
    s1_1 = _spmm(adj1_indices, adj1_values, t1_1)
    s1_2 = _spmm(adj2_indices, adj2_values, t1_2)
    t2_1, t2_2 = _mm2(s1_1, s1_2, W2)
    s2_1 = _spmm(adj1_indices, adj1_values, t2_1)
    s2_2 = _spmm(adj2_indices, adj2_values, t2_2)
    h1, h2, z1, z2, z, label1, label2, x_rec = _heads(
        s2_1, s2_2, W1, W2, att_w1, att_b1, att_w2,
        ins_w1, ins_b1, ins_w2, ins_b2, cls_w1, cls_b1, cls_w2, cls_b2)
    return (h1, h2, z1, z2, z, label1, label2, x_rec)


# speedup vs baseline: 2.2742x; 2.2742x over previous
import functools

import jax
import jax.numpy as jnp
from jax import lax
from jax.experimental import pallas as pl
from jax.experimental.pallas import tpu as pltpu
from jax.experimental.pallas import tpu_sc as plsc

_N = 10000
_E = 320000
_DIN = 128
_DH = 256
_DZ = 64
_NCL = 16
_ATT = 16
_ALPHA = 0.01

_ROWS = 2000


def _elu(x):
    return jnp.where(x > 0, x, jnp.exp(jnp.minimum(x, 0.0)) - 1.0)


def _normalize(x):
    n = jnp.sqrt(jnp.sum(x * x, axis=1, keepdims=True))
    return x / jnp.maximum(n, 1e-12)


def _mm1_body(data_ref, n1_ref, n2_ref, w1_ref, o1_ref, o2_ref):
    w1 = w1_ref[...]
    f1 = data_ref[...] + _ALPHA * n1_ref[...]
    f2 = data_ref[...] + _ALPHA * n2_ref[...]
    o1_ref[...] = jnp.dot(f1, w1, preferred_element_type=jnp.float32)
    o2_ref[...] = jnp.dot(f2, w1, preferred_element_type=jnp.float32)


def _mm1(data, noise1, noise2, W1):
    grid = (_N // _ROWS,)
    return pl.pallas_call(
        _mm1_body,
        grid=grid,
        in_specs=[
            pl.BlockSpec((_ROWS, _DIN), lambda i: (i, 0)),
            pl.BlockSpec((_ROWS, _DIN), lambda i: (i, 0)),
            pl.BlockSpec((_ROWS, _DIN), lambda i: (i, 0)),
            pl.BlockSpec((_DIN, _DH), lambda i: (0, 0)),
        ],
        out_specs=[
            pl.BlockSpec((_ROWS, _DH), lambda i: (i, 0)),
            pl.BlockSpec((_ROWS, _DH), lambda i: (i, 0)),
        ],
        out_shape=[
            jax.ShapeDtypeStruct((_N, _DH), jnp.float32),
            jax.ShapeDtypeStruct((_N, _DH), jnp.float32),
        ],
    )(data, noise1, noise2, W1)


def _mm2_body(s1_ref, s2_ref, w2_ref, o1_ref, o2_ref):
    w2 = w2_ref[...]
    o1_ref[...] = jnp.dot(_elu(s1_ref[...]), w2, preferred_element_type=jnp.float32)
    o2_ref[...] = jnp.dot(_elu(s2_ref[...]), w2, preferred_element_type=jnp.float32)


def _mm2(s1, s2, W2p):
    grid = (_N // _ROWS,)
    return pl.pallas_call(
        _mm2_body,
        grid=grid,
        in_specs=[
            pl.BlockSpec((_ROWS, _DH), lambda i: (i, 0)),
            pl.BlockSpec((_ROWS, _DH), lambda i: (i, 0)),
            pl.BlockSpec((_DH, 2 * _DZ), lambda i: (0, 0)),
        ],
        out_specs=[
            pl.BlockSpec((_ROWS, 2 * _DZ), lambda i: (i, 0)),
            pl.BlockSpec((_ROWS, 2 * _DZ), lambda i: (i, 0)),
        ],
        out_shape=[
            jax.ShapeDtypeStruct((_N, 2 * _DZ), jnp.float32),
            jax.ShapeDtypeStruct((_N, 2 * _DZ), jnp.float32),
        ],
    )(s1, s2, W2p)


def _heads_body(s1_ref, s2_ref, w1_ref, w2_ref,
                aw1_ref, ab1_ref, aw2_ref,
                iw1_ref, ib1_ref, iw2_ref, ib2_ref,
                cw1_ref, cb1_ref, cw2_ref, cb2_ref,
                h1_ref, h2_ref, z1_ref, z2_ref, z_ref,
                l1_ref, l2_ref, xrec_ref):
    z1 = _normalize(_elu(s1_ref[:, :_DZ]))
    z2 = _normalize(_elu(s2_ref[:, :_DZ]))
    z1_ref[...] = z1
    z2_ref[...] = z2

    iw1 = iw1_ref[...]
    ib1 = ib1_ref[...]
    iw2 = iw2_ref[...]
    ib2 = ib2_ref[...]

    def ins_head(z):
        t = jnp.maximum(jnp.dot(z, iw1, preferred_element_type=jnp.float32) + ib1[None, :], 0.0)
        return jnp.maximum(jnp.dot(t, iw2, preferred_element_type=jnp.float32) + ib2[None, :], 0.0)

    h1_ref[...] = _normalize(ins_head(z1))
    h2_ref[...] = _normalize(ins_head(z2))

    cw1 = cw1_ref[...]
    cb1 = cb1_ref[...]
    cw2 = cw2_ref[...]
    cb2 = cb2_ref[...]

    def cls_head(z):
        t = jnp.maximum(jnp.dot(z, cw1, preferred_element_type=jnp.float32) + cb1[None, :], 0.0)
        logit = jnp.dot(t, cw2, preferred_element_type=jnp.float32) + cb2[None, :]
        m = jnp.max(logit, axis=1, keepdims=True)
        e = jnp.exp(logit - m)
        return e / jnp.sum(e, axis=1, keepdims=True)

    l1_ref[...] = cls_head(z1)
    l2_ref[...] = cls_head(z2)

    aw1 = aw1_ref[...]
    ab1 = ab1_ref[...]
    aw2 = aw2_ref[...]
    a1 = jnp.dot(jnp.tanh(jnp.dot(z1, aw1, preferred_element_type=jnp.float32) + ab1[None, :]),
                 aw2, preferred_element_type=jnp.float32)
    a2 = jnp.dot(jnp.tanh(jnp.dot(z2, aw1, preferred_element_type=jnp.float32) + ab1[None, :]),
                 aw2, preferred_element_type=jnp.float32)
    m = jnp.maximum(a1, a2)
    e1 = jnp.exp(a1 - m)
    e2 = jnp.exp(a2 - m)
    denom = e1 + e2
    z = (e1 / denom) * z1 + (e2 / denom) * z2
    z_ref[...] = z

    w1 = w1_ref[...]
    w2 = w2_ref[...]
    t = jnp.maximum(
        jax.lax.dot_general(z, w2, (((1,), (1,)), ((), ())),
                            preferred_element_type=jnp.float32), 0.0)
    xrec_ref[...] = jax.lax.dot_general(t, w1, (((1,), (1,)), ((), ())),
                                        preferred_element_type=jnp.float32)


def _heads(s1, s2, W1, W2, aw1, ab1, aw2, iw1, ib1, iw2, ib2, cw1, cb1, cw2, cb2):
    grid = (_N // _ROWS,)
    row = lambda i: (i, 0)
    zero2 = lambda i: (0, 0)
    zero1 = lambda i: (0,)
    return pl.pallas_call(
        _heads_body,
        grid=grid,
        in_specs=[
            pl.BlockSpec((_ROWS, 2 * _DZ), row),
            pl.BlockSpec((_ROWS, 2 * _DZ), row),
            pl.BlockSpec((_DIN, _DH), zero2),
            pl.BlockSpec((_DH, _DZ), zero2),
            pl.BlockSpec((_DZ, _ATT), zero2),
            pl.BlockSpec((_ATT,), zero1),
            pl.BlockSpec((_ATT, 1), zero2),
            pl.BlockSpec((_DZ, _DZ), zero2),
            pl.BlockSpec((_DZ,), zero1),
            pl.BlockSpec((_DZ, _DZ), zero2),
            pl.BlockSpec((_DZ,), zero1),
            pl.BlockSpec((_DZ, _DZ), zero2),
            pl.BlockSpec((_DZ,), zero1),
            pl.BlockSpec((_DZ, _NCL), zero2),
            pl.BlockSpec((_NCL,), zero1),
        ],
        out_specs=[
            pl.BlockSpec((_ROWS, _DZ), row),
            pl.BlockSpec((_ROWS, _DZ), row),
            pl.BlockSpec((_ROWS, _DZ), row),
            pl.BlockSpec((_ROWS, _DZ), row),
            pl.BlockSpec((_ROWS, _DZ), row),
            pl.BlockSpec((_ROWS, _NCL), row),
            pl.BlockSpec((_ROWS, _NCL), row),
            pl.BlockSpec((_ROWS, _DIN), row),
        ],
        out_shape=[
            jax.ShapeDtypeStruct((_N, _DZ), jnp.float32),
            jax.ShapeDtypeStruct((_N, _DZ), jnp.float32),
            jax.ShapeDtypeStruct((_N, _DZ), jnp.float32),
            jax.ShapeDtypeStruct((_N, _DZ), jnp.float32),
            jax.ShapeDtypeStruct((_N, _DZ), jnp.float32),
            jax.ShapeDtypeStruct((_N, _NCL), jnp.float32),
            jax.ShapeDtypeStruct((_N, _NCL), jnp.float32),
            jax.ShapeDtypeStruct((_N, _DIN), jnp.float32),
        ],
    )(s1, s2, W1, W2, aw1, ab1, aw2, iw1, ib1, iw2, ib2, cw1, cb1, cw2, cb2)


_NSC = 2
_NTILE = 16


_RPT = 624
_TAIL = _N - _RPT * _NTILE
_Dc = 128
_B = 80


def _zero_acc(zh, acc, s):
    pltpu.sync_copy(zh, acc.at[pl.ds(s * _RPT, _RPT)])

    @pl.when(s == 0)
    def _():
        pltpu.sync_copy(zh.at[pl.ds(0, _TAIL)],
                        acc.at[pl.ds(_RPT * _NTILE, _TAIL)])


def _edge_loop(s, xref, srch, dsth, valh, oref,
               srcv, dstv, valv, rows, acc, sem):
    EPT = _E // _NTILE
    nchunks = EPT // _B
    ebase = s * EPT

    def chunk(i, carry):
        e0 = ebase + i * _B
        pltpu.sync_copy(srch.at[pl.ds(e0, _B)], srcv)
        pltpu.sync_copy(dsth.at[pl.ds(e0, _B)], dstv)
        pltpu.sync_copy(valh.at[pl.ds(e0, _B)], valv)
        pltpu.async_copy(xref.at[srcv], rows, sem).wait()

        def scale(k, carry2):
            vv = valv[pl.ds(k * 16, 16)]
            for l in range(16):
                vb = jnp.full((16,), vv[l], jnp.float32)
                e = k * 16 + l
                for j in range(_Dc // 16):
                    sl = pl.ds(j * 16, 16)
                    rows[e, sl] = rows[e, sl] * vb
            return carry2

        lax.fori_loop(0, _B // 16, scale, 0)
        pltpu.sync_copy(rows, acc.at[dstv], add=True)
        return carry

    lax.fori_loop(0, nchunks, chunk, 0)
    plsc.subcore_barrier()
    pltpu.sync_copy(acc.at[pl.ds(s * _RPT, _RPT)],
                    oref.at[pl.ds(s * _RPT, _RPT)])

    @pl.when(s == 0)
    def _():
        pltpu.sync_copy(acc.at[pl.ds(_RPT * _NTILE, _TAIL)],
                        oref.at[pl.ds(_RPT * _NTILE, _TAIL)])


_MESH = plsc.VectorSubcoreMesh(core_axis_name="c", subcore_axis_name="s",
                               num_cores=_NSC, num_subcores=_NTILE)
_SCRATCH = [
    pltpu.VMEM((_B,), jnp.int32),
    pltpu.VMEM((_B,), jnp.int32),
    pltpu.VMEM((_B,), jnp.float32),
    pltpu.VMEM((_B, _Dc), jnp.float32),
    pltpu.VMEM_SHARED((_N, _Dc), jnp.float32),
    pltpu.SemaphoreType.DMA,
]
_OUT2 = [jax.ShapeDtypeStruct((_N, _Dc), jnp.float32),
         jax.ShapeDtypeStruct((_N, _Dc), jnp.float32)]


def _sc_spmm_split_body(xah, xbh, srch, dsth, valh, zh, oah, obh,
                        srcv, dstv, valv, rows, acc, sem):
    c = lax.axis_index("c")
    s = lax.axis_index("s")
    _zero_acc(zh, acc, s)
    plsc.subcore_barrier()

    @pl.when(c == 0)
    def _():
        _edge_loop(s, xah, srch, dsth, valh, oah,
                   srcv, dstv, valv, rows, acc, sem)

    @pl.when(c == 1)
    def _():
        _edge_loop(s, xbh, srch, dsth, valh, obh,
                   srcv, dstv, valv, rows, acc, sem)


_sc_spmm_split = pl.kernel(
    _sc_spmm_split_body, out_type=_OUT2, mesh=_MESH, scratch_types=_SCRATCH)


def _sc_spmm_pair_body(x1h, x2h, src1h, dst1h, val1h, src2h, dst2h, val2h,
                       zh, o1h, o2h, srcv, dstv, valv, rows, acc, sem):
    c = lax.axis_index("c")
    s = lax.axis_index("s")
    _zero_acc(zh, acc, s)
    plsc.subcore_barrier()

    @pl.when(c == 0)
    def _():
        _edge_loop(s, x1h, src1h, dst1h, val1h, o1h,
                   srcv, dstv, valv, rows, acc, sem)

    @pl.when(c == 1)
    def _():
        _edge_loop(s, x2h, src2h, dst2h, val2h, o2h,
                   srcv, dstv, valv, rows, acc, sem)


_sc_spmm_pair = pl.kernel(
    _sc_spmm_pair_body, out_type=_OUT2, mesh=_MESH, scratch_types=_SCRATCH)


def _spmm(idx, vals, X):
    xa = X[:, :_Dc]
    xb = X[:, _Dc:]
    z = jnp.zeros((_RPT, _Dc), jnp.float32)
    oa, ob = _sc_spmm_split(xa, xb, idx[1], idx[0], vals, z)
    return jnp.concatenate([oa, ob], axis=1)


def _spmm_pair(idx1, vals1, X1, idx2, vals2, X2):
    z = jnp.zeros((_RPT, _Dc), jnp.float32)
    return _sc_spmm_pair(X1, X2, idx1[1], idx1[0], vals1,
                         idx2[1], idx2[0], vals2, z)


def kernel(data, adj1_indices, adj1_values, adj2_indices, adj2_values,
           W1, W2, att_w1, att_b1, att_w2,
           ins_w1, ins_b1, ins_w2, ins_b2,
           cls_w1, cls_b1, cls_w2, cls_b2):
    noise1 = jax.random.normal(jax.random.key(101), data.shape, jnp.float32)
    noise2 = jax.random.normal(jax.random.key(202), data.shape, jnp.float32)

    t1_1, t1_2 = _mm1(data, noise1, noise2, W1)
    s1_1 = _spmm(adj1_indices, adj1_values, t1_1)
    s1_2 = _spmm(adj2_indices, adj2_values, t1_2)
    W2p = jnp.concatenate([W2, jnp.zeros((_DH, _DZ), jnp.float32)], axis=1)
    t2_1, t2_2 = _mm2(s1_1, s1_2, W2p)
    s2_1, s2_2 = _spmm_pair(adj1_indices, adj1_values, t2_1,
                            adj2_indices, adj2_values, t2_2)
    h1, h2, z1, z2, z, label1, label2, x_rec = _heads(
        s2_1, s2_2, W1, W2, att_w1, att_b1, att_w2,
        ins_w1, ins_b1, ins_w2, ins_b2, cls_w1, cls_b1, cls_w2, cls_b2)
    return (h1, h2, z1, z2, z, label1, label2, x_rec)
